# Initial kernel scaffold; baseline (speedup 1.0000x reference)
import jax
import jax.numpy as jnp
from jax.experimental import pallas as pl


def kernel(alleles, positions, allele_table, kernel_table, bias_table):
    raise NotImplementedError("write your pallas kernel here")


# speedup vs baseline: 4.7632x; 4.7632x over previous
import functools

import jax
import jax.numpy as jnp
from jax import lax
from jax.experimental import pallas as pl
from jax.experimental.pallas import tpu as pltpu
from jax.experimental.pallas import tpu_sc as plsc

D = 16
CH = 64


def _sc_body(all_hbm, pos_hbm, at_hbm, kt_hbm, bt_hbm, out_hbm,
             posv, allv, krows, biasv, arows, outb,
             sem_k, sem_b, sem_a, *, n_per_w, nc):
    wid = lax.axis_index("s") * nc + lax.axis_index("c")
    base0 = wid * n_per_w
    nchunk = n_per_w // CH

    def compute_pos(j, carry):
        a0 = arows[2 * j, :]
        a1 = arows[2 * j + 1, :]
        av = a0 + a1
        acc0 = biasv[j, :]
        acc1 = jnp.zeros((D,), jnp.float32)
        acc2 = jnp.zeros((D,), jnp.float32)
        acc3 = jnp.zeros((D,), jnp.float32)
        acc = [acc0, acc1, acc2, acc3]
        for i in range(D):
            s = av[i]
            acc[i % 4] = acc[i % 4] + s * krows[j, pl.ds(i * D, D)]
        outb[j, :] = (acc[0] + acc[1]) + (acc[2] + acc[3])
        return carry

    def chunk(g, carry):
        base = base0 + g * CH
        pltpu.sync_copy(pos_hbm.at[pl.ds(base, CH)], posv)
        pltpu.sync_copy(all_hbm.at[pl.ds(2 * base, 2 * CH)], allv)
        ck = pltpu.make_async_copy(kt_hbm.at[posv], krows, sem_k)
        cb = pltpu.make_async_copy(bt_hbm.at[posv], biasv, sem_b)
        ca = pltpu.make_async_copy(at_hbm.at[allv], arows, sem_a)
        ck.start()
        cb.start()
        ca.start()
        ck.wait()
        cb.wait()
        ca.wait()
        lax.fori_loop(0, CH, compute_pos, 0, unroll=False)
        pltpu.sync_copy(outb, out_hbm.at[pl.ds(base, CH)])
        return carry

    lax.fori_loop(0, nchunk, chunk, 0, unroll=False)


@functools.partial(jax.jit, static_argnames=())
def _run(alleles_flat, positions_flat, allele_table, kernel_table, bias_table):
    n = positions_flat.shape[0]
    info = plsc.get_sparse_core_info()
    nc, ns = info.num_cores, info.num_subcores
    nw = nc * ns
    n_per_w = n // nw
    mesh = plsc.VectorSubcoreMesh(core_axis_name="c", subcore_axis_name="s")
    kern = pl.kernel(
        functools.partial(_sc_body, n_per_w=n_per_w, nc=nc),
        out_type=jax.ShapeDtypeStruct((n, D), jnp.float32),
        mesh=mesh,
        compiler_params=pltpu.CompilerParams(use_tc_tiling_on_sc=False),
        scratch_types=[
            pltpu.VMEM((CH,), jnp.int32),
            pltpu.VMEM((2 * CH,), jnp.int32),
            pltpu.VMEM((CH, D * D), jnp.float32),
            pltpu.VMEM((CH, D), jnp.float32),
            pltpu.VMEM((2 * CH, D), jnp.float32),
            pltpu.VMEM((CH, D), jnp.float32),
            pltpu.SemaphoreType.DMA,
            pltpu.SemaphoreType.DMA,
            pltpu.SemaphoreType.DMA,
        ],
    )
    return kern(alleles_flat, positions_flat, allele_table, kernel_table,
                bias_table)


def kernel(alleles, positions, allele_table, kernel_table, bias_table):
    b, p, ploidy = alleles.shape
    n = b * p
    af = alleles.reshape(n * ploidy).astype(jnp.int32)
    pf = positions.reshape(n).astype(jnp.int32)
    out = _run(af, pf, allele_table, kernel_table, bias_table)
    return out.reshape(b, p, D)


# speedup vs baseline: 7.0018x; 1.4700x over previous
import functools

import jax
import jax.numpy as jnp
from jax import lax
from jax.experimental import pallas as pl
from jax.experimental.pallas import tpu as pltpu
from jax.experimental.pallas import tpu_sc as plsc

D = 16
CH = 128
NBUF = 2


def _sc_body(all_hbm, pos_hbm, at_hbm, kt_hbm, bt_hbm, out_hbm,
             posv0, posv1, ala0, ala1, alb0, alb1,
             krows0, krows1, biasv0, biasv1, arows0, arows1, outb0, outb1,
             semi0, semi1, semd0, semd1, semo0, semo1, *, n_per_w, nc):
    wid = lax.axis_index("s") * nc + lax.axis_index("c")
    base0 = wid * n_per_w
    nchunk = n_per_w // CH

    posv = [posv0, posv1]
    ala = [ala0, ala1]
    alb = [alb0, alb1]
    krows = [krows0, krows1]
    biasv = [biasv0, biasv1]
    arows = [arows0, arows1]
    outb = [outb0, outb1]
    semi = [semi0, semi1]
    semd = [semd0, semd1]
    semo = [semo0, semo1]

    def idx_copies(g, s):
        base = base0 + g * CH
        return [
            pltpu.make_async_copy(pos_hbm.at[pl.ds(base, CH)], posv[s],
                                  semi[s]),
            pltpu.make_async_copy(all_hbm.at[pl.ds(2 * base, CH)], ala[s],
                                  semi[s]),
            pltpu.make_async_copy(all_hbm.at[pl.ds(2 * base + CH, CH)],
                                  alb[s], semi[s]),
        ]

    def dat_copies(s):
        return [
            pltpu.make_async_copy(kt_hbm.at[posv[s]], krows[s], semd[s]),
            pltpu.make_async_copy(bt_hbm.at[posv[s]], biasv[s], semd[s]),
            pltpu.make_async_copy(at_hbm.at[ala[s]],
                                  arows[s].at[pl.ds(0, CH)], semd[s]),
            pltpu.make_async_copy(at_hbm.at[alb[s]],
                                  arows[s].at[pl.ds(CH, CH)], semd[s]),
        ]

    def out_copy(g, s):
        base = base0 + g * CH
        return pltpu.make_async_copy(outb[s], out_hbm.at[pl.ds(base, CH)],
                                     semo[s])

    def one_pos(s, j):
        a0 = arows[s][2 * j, :]
        a1 = arows[s][2 * j + 1, :]
        av = a0 + a1
        acc = [biasv[s][j, :],
               jnp.zeros((D,), jnp.float32),
               jnp.zeros((D,), jnp.float32),
               jnp.zeros((D,), jnp.float32)]
        for i in range(D):
            acc[i % 4] = acc[i % 4] + av[i] * krows[s][j, pl.ds(i * D, D)]
        outb[s][j, :] = (acc[0] + acc[1]) + (acc[2] + acc[3])

    def compute(s):
        def body(t, carry):
            one_pos(s, 2 * t)
            one_pos(s, 2 * t + 1)
            return carry
        lax.fori_loop(0, CH // 2, body, 0, unroll=False)

    def chunk_step(g, s, *, fetch_idx, issue_next, wait_out):
        s1 = 1 - s
        if issue_next:
            for c in idx_copies(g + 1, s1):
                c.wait()
            for c in dat_copies(s1):
                c.start()
        for c in dat_copies(s):
            c.wait()
        if fetch_idx:
            for c in idx_copies(g + 2, s):
                c.start()
        if wait_out:
            out_copy(g, s).wait()
        compute(s)
        out_copy(g, s).start()

    for c in idx_copies(0, 0):
        c.start()
    for c in idx_copies(0, 0):
        c.wait()
    for c in dat_copies(0):
        c.start()
    for c in idx_copies(1, 1):
        c.start()

    chunk_step(0, 0, fetch_idx=True, issue_next=True, wait_out=False)
    chunk_step(1, 1, fetch_idx=True, issue_next=True, wait_out=False)

    def main_body(gg, carry):
        g = 2 * gg
        chunk_step(g, 0, fetch_idx=True, issue_next=True, wait_out=True)
        chunk_step(g + 1, 1, fetch_idx=True, issue_next=True, wait_out=True)
        return carry
    lax.fori_loop(1, nchunk // 2 - 1, main_body, 0, unroll=False)

    chunk_step(nchunk - 2, 0, fetch_idx=False, issue_next=True, wait_out=True)
    chunk_step(nchunk - 1, 1, fetch_idx=False, issue_next=False,
               wait_out=True)
    out_copy(nchunk - 2, 0).wait()
    out_copy(nchunk - 1, 1).wait()


@functools.partial(jax.jit, static_argnames=())
def _run(alleles_flat, positions_flat, allele_table, kernel_table, bias_table):
    n = positions_flat.shape[0]
    info = plsc.get_sparse_core_info()
    nc, ns = info.num_cores, info.num_subcores
    nw = nc * ns
    n_per_w = n // nw
    mesh = plsc.VectorSubcoreMesh(core_axis_name="c", subcore_axis_name="s")
    kern = pl.kernel(
        functools.partial(_sc_body, n_per_w=n_per_w, nc=nc),
        out_type=jax.ShapeDtypeStruct((n, D), jnp.float32),
        mesh=mesh,
        compiler_params=pltpu.CompilerParams(use_tc_tiling_on_sc=False),
        scratch_types=(
            [pltpu.VMEM((CH,), jnp.int32)] * 2 +
            [pltpu.VMEM((CH,), jnp.int32)] * 2 +
            [pltpu.VMEM((CH,), jnp.int32)] * 2 +
            [pltpu.VMEM((CH, D * D), jnp.float32)] * 2 +
            [pltpu.VMEM((CH, D), jnp.float32)] * 2 +
            [pltpu.VMEM((2 * CH, D), jnp.float32)] * 2 +
            [pltpu.VMEM((CH, D), jnp.float32)] * 2 +
            [pltpu.SemaphoreType.DMA] * 6
        ),
    )
    return kern(alleles_flat, positions_flat, allele_table, kernel_table,
                bias_table)


def kernel(alleles, positions, allele_table, kernel_table, bias_table):
    b, p, ploidy = alleles.shape
    n = b * p
    af = alleles.reshape(n * ploidy).astype(jnp.int32)
    pf = positions.reshape(n).astype(jnp.int32)
    out = _run(af, pf, allele_table, kernel_table, bias_table)
    return out.reshape(b, p, D)


# speedup vs baseline: 7.0835x; 1.0117x over previous
import functools

import jax
import jax.numpy as jnp
from jax import lax
from jax.experimental import pallas as pl
from jax.experimental.pallas import tpu as pltpu
from jax.experimental.pallas import tpu_sc as plsc

D = 16
CH = 80
NBUF = 4


def _sc_body(all_hbm, pos_hbm, at_hbm, kt_hbm, bt_hbm, out_hbm,
             *scratch, n_per_w, nc):
    wid = lax.axis_index("s") * nc + lax.axis_index("c")
    base0 = wid * n_per_w
    nchunk = n_per_w // CH

    posv = scratch[0 * NBUF:1 * NBUF]
    ala = scratch[1 * NBUF:2 * NBUF]
    alb = scratch[2 * NBUF:3 * NBUF]
    krows = scratch[3 * NBUF:4 * NBUF]
    biasv = scratch[4 * NBUF:5 * NBUF]
    arows = scratch[5 * NBUF:6 * NBUF]
    outb = scratch[6 * NBUF:7 * NBUF]
    semi = scratch[7 * NBUF:8 * NBUF]
    semd = scratch[8 * NBUF:9 * NBUF]
    semo = scratch[9 * NBUF:10 * NBUF]

    def idx_copies(g, s):
        base = base0 + g * CH
        return [
            pltpu.make_async_copy(pos_hbm.at[pl.ds(base, CH)], posv[s],
                                  semi[s]),
            pltpu.make_async_copy(all_hbm.at[pl.ds(2 * base, CH)], ala[s],
                                  semi[s]),
            pltpu.make_async_copy(all_hbm.at[pl.ds(2 * base + CH, CH)],
                                  alb[s], semi[s]),
        ]

    def dat_copies(s):
        return [
            pltpu.make_async_copy(kt_hbm.at[posv[s]], krows[s], semd[s]),
            pltpu.make_async_copy(bt_hbm.at[posv[s]], biasv[s], semd[s]),
            pltpu.make_async_copy(at_hbm.at[ala[s]],
                                  arows[s].at[pl.ds(0, CH)], semd[s]),
            pltpu.make_async_copy(at_hbm.at[alb[s]],
                                  arows[s].at[pl.ds(CH, CH)], semd[s]),
        ]

    def out_copy(g, s):
        base = base0 + g * CH
        return pltpu.make_async_copy(outb[s], out_hbm.at[pl.ds(base, CH)],
                                     semo[s])

    def one_pos(s, j):
        a0 = arows[s][2 * j, :]
        a1 = arows[s][2 * j + 1, :]
        av = a0 + a1
        acc = [biasv[s][j, :],
               jnp.zeros((D,), jnp.float32),
               jnp.zeros((D,), jnp.float32),
               jnp.zeros((D,), jnp.float32)]
        for i in range(D):
            acc[i % 4] = acc[i % 4] + av[i] * krows[s][j, pl.ds(i * D, D)]
        outb[s][j, :] = (acc[0] + acc[1]) + (acc[2] + acc[3])

    def compute(s):
        def body(t, carry):
            one_pos(s, 2 * t)
            one_pos(s, 2 * t + 1)
            return carry
        lax.fori_loop(0, CH // 2, body, 0, unroll=False)

    def step(g, s, *, fetch, issue, wait_out):
        for c in dat_copies(s):
            c.wait()
        if fetch:
            for c in idx_copies(g + NBUF, s):
                c.start()
        if issue:
            s_prev = (s - 1) % NBUF
            for c in idx_copies(g + NBUF - 1, s_prev):
                c.wait()
            for c in dat_copies(s_prev):
                c.start()
        if wait_out:
            out_copy(g, s).wait()
        compute(s)
        out_copy(g, s).start()

    for g in range(NBUF):
        for c in idx_copies(g, g):
            c.start()
    for g in range(NBUF - 1):
        for c in idx_copies(g, g):
            c.wait()
        for c in dat_copies(g):
            c.start()

    for g in range(NBUF):
        step(g, g, fetch=True, issue=True, wait_out=False)

    def main_body(gg, carry):
        g0 = gg * NBUF
        for b in range(NBUF):
            step(g0 + b, b, fetch=True, issue=True, wait_out=True)
        return carry
    lax.fori_loop(1, nchunk // NBUF - 1, main_body, 0, unroll=False)

    for b in range(NBUF):
        g = nchunk - NBUF + b
        step(g, b, fetch=False, issue=(b == 0), wait_out=True)
    for b in range(NBUF):
        out_copy(nchunk - NBUF + b, b).wait()


@functools.partial(jax.jit, static_argnames=())
def _run(alleles_flat, positions_flat, allele_table, kernel_table, bias_table):
    n = positions_flat.shape[0]
    info = plsc.get_sparse_core_info()
    nc, ns = info.num_cores, info.num_subcores
    nw = nc * ns
    n_per_w = n // nw
    mesh = plsc.VectorSubcoreMesh(core_axis_name="c", subcore_axis_name="s")
    kern = pl.kernel(
        functools.partial(_sc_body, n_per_w=n_per_w, nc=nc),
        out_type=jax.ShapeDtypeStruct((n, D), jnp.float32),
        mesh=mesh,
        compiler_params=pltpu.CompilerParams(use_tc_tiling_on_sc=False),
        scratch_types=(
            [pltpu.VMEM((CH,), jnp.int32)] * NBUF +
            [pltpu.VMEM((CH,), jnp.int32)] * NBUF +
            [pltpu.VMEM((CH,), jnp.int32)] * NBUF +
            [pltpu.VMEM((CH, D * D), jnp.float32)] * NBUF +
            [pltpu.VMEM((CH, D), jnp.float32)] * NBUF +
            [pltpu.VMEM((2 * CH, D), jnp.float32)] * NBUF +
            [pltpu.VMEM((CH, D), jnp.float32)] * NBUF +
            [pltpu.SemaphoreType.DMA] * (3 * NBUF)
        ),
    )
    return kern(alleles_flat, positions_flat, allele_table, kernel_table,
                bias_table)


def kernel(alleles, positions, allele_table, kernel_table, bias_table):
    b, p, ploidy = alleles.shape
    n = b * p
    af = alleles.reshape(n * ploidy).astype(jnp.int32)
    pf = positions.reshape(n).astype(jnp.int32)
    out = _run(af, pf, allele_table, kernel_table, bias_table)
    return out.reshape(b, p, D)
